# baseline (speedup 1.0000x reference)
import functools

import jax
import jax.numpy as jnp
from jax import lax
from jax.experimental import pallas as pl
from jax.experimental.pallas import tpu as pltpu
from jax.experimental.pallas import tpu_sc as plsc

N = 50000
G = 64
H = 64
GE = 64
C = 5
IN_CH = 2

NC = 2
NS = 16
NP = 50176
RPT = NP // NS
STG = 784

CHUNK = 1024
ECHUNK = 512
E = 800000
TOT_CHUNKS = 800
E_PAD = TOT_CHUNKS * CHUNK
CPT = TOT_CHUNKS // NS
CPT_HALF = TOT_CHUNKS // (NS * NC)

DF = H // 4

RB = RPT
GRID = NP // RB

_HIGH = lax.Precision.HIGHEST


def _mesh():
    return plsc.VectorSubcoreMesh(core_axis_name="c", subcore_axis_name="s",
                                  num_cores=NC, num_subcores=NS)


def _deg_body(dst_c, deg2, idx_d, ones_v, stage, acc, ssem):
    c = lax.axis_index("c")
    s = lax.axis_index("s")
    r0 = s * RPT
    for j in range(STG // 16):
        stage[pl.ds(j * 16, 16)] = jnp.zeros((16,), jnp.float32)
    for j in range(CHUNK // 16):
        ones_v[pl.ds(j * 16, 16)] = jnp.ones((16,), jnp.float32)
    for q in range(RPT // STG):
        pltpu.sync_copy(stage, acc.at[pl.ds(r0 + q * STG, STG)])
    plsc.subcore_barrier()
    base = c * (TOT_CHUNKS // NC) + s * CPT_HALF

    def emit_block(b, p, drain):
        if drain:
            pltpu.make_async_copy(deg2.at[pl.ds(0, CHUNK)], ones_v,
                                  ssem).wait()
        pltpu.sync_copy(dst_c.at[base + b], idx_d.at[p])
        pltpu.async_copy(ones_v, acc.at[idx_d.at[p]], ssem, add=True)

    emit_block(0, 0, False)
    emit_block(1, 1, False)

    def blk(b, carry):
        emit_block(b, lax.rem(b, 2), True)
        return carry

    lax.fori_loop(2, CPT_HALF, blk, 0)
    for p in range(2):
        pltpu.make_async_copy(deg2.at[pl.ds(0, CHUNK)], ones_v,
                              ssem).wait()
    plsc.subcore_barrier()
    for q in range(RPT // STG):
        pltpu.sync_copy(acc.at[pl.ds(r0 + q * STG, STG)], stage)
        pltpu.sync_copy(stage, deg2.at[pl.ds(c * NP + r0 + q * STG, STG)])


@functools.lru_cache(maxsize=None)
def _deg_pass():
    return pl.kernel(
        _deg_body,
        out_type=jax.ShapeDtypeStruct((NC * NP,), jnp.float32),
        mesh=_mesh(),
        scratch_types=[
            pltpu.VMEM((2, CHUNK), jnp.int32),
            pltpu.VMEM((CHUNK,), jnp.float32),
            pltpu.VMEM((STG,), jnp.float32),
            pltpu.VMEM_SHARED((NP,), jnp.float32),
            pltpu.SemaphoreType.DMA,
        ],
        compiler_params=pltpu.CompilerParams(use_tc_tiling_on_sc=False),
    )


def _edge_pass_body(D, edge_split, chunk):
    def body(src_c, dst_c, gi, so, idx_s, idx_d, rows, stage, acc, gsem,
             ssem):
        c = lax.axis_index("c")
        s = lax.axis_index("s")
        r0 = s * RPT
        for q in range(RPT // STG):
            pltpu.sync_copy(gi.at[c, pl.ds(r0 + q * STG, STG)], stage)
            pltpu.sync_copy(stage, acc.at[pl.ds(r0 + q * STG, STG)])
        plsc.subcore_barrier()
        tot = E_PAD // chunk
        if edge_split:
            nblk = tot // (NS * NC)
            base = c * (tot // NC) + s * nblk
            table = gi.at[0]
        else:
            nblk = tot // NS
            base = s * nblk
            table = gi.at[c]

        def emit_block(b, p, drain):
            if drain:
                pltpu.make_async_copy(gi.at[0, pl.ds(0, chunk)],
                                      rows.at[p], ssem).wait()
            pltpu.sync_copy(src_c.at[base + b], idx_s.at[p])
            pltpu.sync_copy(dst_c.at[base + b], idx_d.at[p])
            pltpu.async_copy(table.at[idx_s.at[p]], rows.at[p],
                             gsem).wait()
            pltpu.async_copy(rows.at[p], acc.at[idx_d.at[p]], ssem,
                             add=True)

        emit_block(0, 0, False)
        emit_block(1, 1, False)

        def blk(b, carry):
            emit_block(b, lax.rem(b, 2), True)
            return carry

        lax.fori_loop(2, nblk, blk, 0)
        for p in range(2):
            pltpu.make_async_copy(gi.at[0, pl.ds(0, chunk)],
                                  rows.at[p], ssem).wait()
        plsc.subcore_barrier()
        for q in range(RPT // STG):
            pltpu.sync_copy(acc.at[pl.ds(r0 + q * STG, STG)], stage)
            pltpu.sync_copy(stage, so.at[c, pl.ds(r0 + q * STG, STG)])
    return body


@functools.lru_cache(maxsize=None)
def _make_edge_pass(D, edge_split, chunk=None):
    if chunk is None:
        chunk = ECHUNK
    return pl.kernel(
        _edge_pass_body(D, edge_split, chunk),
        out_type=jax.ShapeDtypeStruct((NC, NP, D), jnp.float32),
        mesh=_mesh(),
        scratch_types=[
            pltpu.VMEM((2, chunk), jnp.int32),
            pltpu.VMEM((2, chunk), jnp.int32),
            pltpu.VMEM((2, chunk, D), jnp.float32),
            pltpu.VMEM((STG, D), jnp.float32),
            pltpu.VMEM_SHARED((NP, D), jnp.float32),
            pltpu.SemaphoreType.DMA,
            pltpu.SemaphoreType.DMA,
        ],
        compiler_params=pltpu.CompilerParams(use_tc_tiling_on_sc=False),
    )


def _prep_body(deg_ref, x_ref, dinv_ref, g1i_ref):
    d = deg_ref[0] + deg_ref[1] + 1.0
    dinv = lax.rsqrt(jnp.maximum(d, 1.0))
    dinv_ref[...] = dinv
    g1i_ref[0] = jnp.concatenate(
        [x_ref[...] * dinv, jnp.zeros((RB, DF - IN_CH), jnp.float32)],
        axis=1)
    g1i_ref[1] = jnp.zeros((RB, DF), jnp.float32)


_prep = pl.pallas_call(
    _prep_body,
    grid=(GRID,),
    in_specs=[pl.BlockSpec((NC, RB, 1), lambda i: (0, i, 0)),
              pl.BlockSpec((RB, IN_CH), lambda i: (i, 0))],
    out_specs=[pl.BlockSpec((RB, 1), lambda i: (i, 0)),
               pl.BlockSpec((NC, RB, DF), lambda i: (0, i, 0))],
    out_shape=[jax.ShapeDtypeStruct((NP, 1), jnp.float32),
               jax.ShapeDtypeStruct((NC, NP, DF), jnp.float32)],
)


def _layer_body(mode):
    def body(*refs):
        if mode == "sum":
            s_ref, dinv_ref, w_ref, b_ref, ga_ref, gb_ref = refs
            sv = (s_ref[0] + s_ref[1])[:, :IN_CH]
        else:
            sa_ref, sb_ref, dinv_ref, w_ref, b_ref, ga_ref, gb_ref = refs
            sv = jnp.concatenate([sa_ref[0], sa_ref[1], sb_ref[0],
                                  sb_ref[1]], axis=1)
        dinv = dinv_ref[...]
        h = jnp.dot(sv * dinv, w_ref[...], precision=_HIGH,
                    preferred_element_type=jnp.float32) + b_ref[...]
        h = jnp.maximum(h, 0.0)
        gv = h * dinv
        ga_ref[0] = gv[:, 0 * DF:1 * DF]
        ga_ref[1] = gv[:, 1 * DF:2 * DF]
        gb_ref[0] = gv[:, 2 * DF:3 * DF]
        gb_ref[1] = gv[:, 3 * DF:4 * DF]
    return body


def _make_layer(mode, din):
    s_specs = ([pl.BlockSpec((NC, RB, DF), lambda i: (0, i, 0))]
               if mode == "sum" else
               [pl.BlockSpec((NC, RB, DF), lambda i: (0, i, 0)),
                pl.BlockSpec((NC, RB, DF), lambda i: (0, i, 0))])
    return pl.pallas_call(
        _layer_body(mode),
        grid=(GRID,),
        in_specs=s_specs + [
            pl.BlockSpec((RB, 1), lambda i: (i, 0)),
            pl.BlockSpec((din, H), lambda i: (0, 0)),
            pl.BlockSpec((1, H), lambda i: (0, 0))],
        out_specs=[pl.BlockSpec((NC, RB, DF), lambda i: (0, i, 0)),
                   pl.BlockSpec((NC, RB, DF), lambda i: (0, i, 0))],
        out_shape=[jax.ShapeDtypeStruct((NC, NP, DF), jnp.float32),
                   jax.ShapeDtypeStruct((NC, NP, DF), jnp.float32)],
    )


_layer1 = _make_layer("sum", IN_CH)
_layer2 = _make_layer("concat", H)


def _final_body(sa_ref, sb_ref, dinv_ref, w_ref, b_ref, batch_ref, ge_ref,
                wl_ref, bl_ref, out_ref, pooled, counts):
    i = pl.program_id(0)

    @pl.when(i == 0)
    def _():
        pooled[...] = jnp.zeros((G, H), jnp.float32)
        counts[...] = jnp.zeros((G, 1), jnp.float32)

    sv = jnp.concatenate([sa_ref[0], sa_ref[1], sb_ref[0], sb_ref[1]],
                         axis=1)
    h3 = jnp.dot(sv * dinv_ref[...], w_ref[...], precision=_HIGH,
                 preferred_element_type=jnp.float32) + b_ref[...]
    oh = (batch_ref[...] == lax.broadcasted_iota(jnp.int32, (RB, G), 1)
          ).astype(jnp.float32)
    pooled[...] += lax.dot_general(oh, h3, (((0,), (0,)), ((), ())),
                                   precision=_HIGH,
                                   preferred_element_type=jnp.float32)
    counts[...] += lax.dot_general(oh, jnp.ones((RB, 1), jnp.float32),
                                   (((0,), (0,)), ((), ())),
                                   precision=_HIGH,
                                   preferred_element_type=jnp.float32)

    @pl.when(i == GRID - 1)
    def _():
        pm = pooled[...] / jnp.maximum(counts[...], 1.0)
        z = jnp.dot(jnp.concatenate([pm, ge_ref[...]], axis=1), wl_ref[...],
                    precision=_HIGH,
                    preferred_element_type=jnp.float32) + bl_ref[...]
        m = jnp.max(z, axis=1, keepdims=True)
        lse = jnp.log(jnp.sum(jnp.exp(z - m), axis=1, keepdims=True)) + m
        out_ref[...] = z - lse


_final = pl.pallas_call(
    _final_body,
    grid=(GRID,),
    in_specs=[pl.BlockSpec((NC, RB, DF), lambda i: (0, i, 0)),
              pl.BlockSpec((NC, RB, DF), lambda i: (0, i, 0)),
              pl.BlockSpec((RB, 1), lambda i: (i, 0)),
              pl.BlockSpec((H, H), lambda i: (0, 0)),
              pl.BlockSpec((1, H), lambda i: (0, 0)),
              pl.BlockSpec((RB, 1), lambda i: (i, 0)),
              pl.BlockSpec((G, GE), lambda i: (0, 0)),
              pl.BlockSpec((H + GE, C), lambda i: (0, 0)),
              pl.BlockSpec((1, C), lambda i: (0, 0))],
    out_specs=pl.BlockSpec((G, C), lambda i: (0, 0)),
    out_shape=jax.ShapeDtypeStruct((G, C), jnp.float32),
    scratch_shapes=[pltpu.VMEM((G, H), jnp.float32),
                    pltpu.VMEM((G, 1), jnp.float32)],
)


def kernel(x, edge_index, batch, graph_embedding, W1, b1, W2, b2, W3, b3,
           Wl, bl):
    src = edge_index[0].astype(jnp.int32)
    dst = edge_index[1].astype(jnp.int32)
    pad_e = E_PAD - E
    src_p = jnp.concatenate([src, jnp.zeros((pad_e,), jnp.int32)])
    dst_p = jnp.concatenate([dst, jnp.full((pad_e,), N, jnp.int32)])
    src_c = src_p.reshape(E_PAD // ECHUNK, ECHUNK)
    dst_c = dst_p.reshape(E_PAD // ECHUNK, ECHUNK)
    dst_cd = dst_p.reshape(TOT_CHUNKS, CHUNK)
    x_pad = jnp.pad(x, ((0, NP - N), (0, 0)))
    batch_pad = jnp.pad(batch.astype(jnp.int32), (0, NP - N),
                        constant_values=G).reshape(NP, 1)
    deg2 = _deg_pass()(dst_cd)
    dinv, g1i = _prep(deg2.reshape(NC, NP, 1), x_pad)
    pass1 = _make_edge_pass(DF, True)
    passf = _make_edge_pass(DF, False)
    s1 = pass1(src_c, dst_c, g1i)
    g2a, g2b = _layer1(s1, dinv, W1, b1.reshape(1, H))
    s2a = passf(src_c, dst_c, g2a)
    s2b = passf(src_c, dst_c, g2b)
    g3a, g3b = _layer2(s2a, s2b, dinv, W2, b2.reshape(1, H))
    s3a = passf(src_c, dst_c, g3a)
    s3b = passf(src_c, dst_c, g3b)
    return _final(s3a, s3b, dinv, W3, b3.reshape(1, H), batch_pad,
                  graph_embedding, Wl, bl.reshape(1, C))


# speedup vs baseline: 1.1897x; 1.1897x over previous
import functools

import jax
import jax.numpy as jnp
from jax import lax
from jax.experimental import pallas as pl
from jax.experimental.pallas import tpu as pltpu
from jax.experimental.pallas import tpu_sc as plsc

N = 50000
G = 64
H = 64
GE = 64
C = 5
IN_CH = 2

NC = 2
NS = 16
NP = 50176
RPT = NP // NS
STG = 784

CHUNK = 1024
ECHUNK = 128
EBLK = 10
E = 800000
TOT_CHUNKS = 800
E_PAD = TOT_CHUNKS * CHUNK
CPT = TOT_CHUNKS // NS
CPT_HALF = TOT_CHUNKS // (NS * NC)

DF = H // 4

RB = RPT
GRID = NP // RB

_HIGH = lax.Precision.HIGHEST


def _mesh():
    return plsc.VectorSubcoreMesh(core_axis_name="c", subcore_axis_name="s",
                                  num_cores=NC, num_subcores=NS)


def _deg_body(dst_c, deg2, idx_d, ones_v, stage, acc, ssem):
    c = lax.axis_index("c")
    s = lax.axis_index("s")
    r0 = s * RPT
    for j in range(STG // 16):
        stage[pl.ds(j * 16, 16)] = jnp.zeros((16,), jnp.float32)
    for j in range(CHUNK // 16):
        ones_v[pl.ds(j * 16, 16)] = jnp.ones((16,), jnp.float32)
    for q in range(RPT // STG):
        pltpu.sync_copy(stage, acc.at[pl.ds(r0 + q * STG, STG)])
    plsc.subcore_barrier()
    base = c * (TOT_CHUNKS // NC) + s * CPT_HALF

    def emit_block(b, p, drain):
        if drain:
            pltpu.make_async_copy(deg2.at[pl.ds(0, CHUNK)], ones_v,
                                  ssem).wait()
        pltpu.sync_copy(dst_c.at[base + b], idx_d.at[p])
        pltpu.async_copy(ones_v, acc.at[idx_d.at[p]], ssem, add=True)

    emit_block(0, 0, False)
    emit_block(1, 1, False)

    def blk(b, carry):
        emit_block(b, lax.rem(b, 2), True)
        return carry

    lax.fori_loop(2, CPT_HALF, blk, 0)
    for p in range(2):
        pltpu.make_async_copy(deg2.at[pl.ds(0, CHUNK)], ones_v,
                              ssem).wait()
    plsc.subcore_barrier()
    for q in range(RPT // STG):
        pltpu.sync_copy(acc.at[pl.ds(r0 + q * STG, STG)], stage)
        pltpu.sync_copy(stage, deg2.at[pl.ds(c * NP + r0 + q * STG, STG)])


@functools.lru_cache(maxsize=None)
def _deg_pass():
    return pl.kernel(
        _deg_body,
        out_type=jax.ShapeDtypeStruct((NC * NP,), jnp.float32),
        mesh=_mesh(),
        scratch_types=[
            pltpu.VMEM((2, CHUNK), jnp.int32),
            pltpu.VMEM((CHUNK,), jnp.float32),
            pltpu.VMEM((STG,), jnp.float32),
            pltpu.VMEM_SHARED((NP,), jnp.float32),
            pltpu.SemaphoreType.DMA,
        ],
        compiler_params=pltpu.CompilerParams(use_tc_tiling_on_sc=False),
    )


def _edge_pass_body(D, edge_split, blk, chunk):
    def body(src_c, dst_c, gi, so, idx_s, idx_d, rows, stage, acc, gsem,
             ssem):
        c = lax.axis_index("c")
        s = lax.axis_index("s")
        r0 = s * RPT
        for q in range(RPT // STG):
            pltpu.sync_copy(gi.at[c, pl.ds(r0 + q * STG, STG)], stage)
            pltpu.sync_copy(stage, acc.at[pl.ds(r0 + q * STG, STG)])
        plsc.subcore_barrier()
        tot = E_PAD // chunk
        if edge_split:
            nblk = tot // (NS * NC * blk)
            base = c * (tot // NC) + s * nblk * blk
            table = gi.at[0]
        else:
            nblk = tot // (NS * blk)
            base = s * nblk * blk
            table = gi.at[c]

        def emit_block(b, p, drain):
            if drain:
                pltpu.make_async_copy(gi.at[0, pl.ds(0, blk * chunk)],
                                      rows.at[p], ssem).wait()
            pltpu.sync_copy(src_c.at[pl.ds((base + b * blk), blk)],
                            idx_s.at[p])
            pltpu.sync_copy(dst_c.at[pl.ds((base + b * blk), blk)],
                            idx_d.at[p])
            gds = [pltpu.async_copy(
                       table.at[idx_s.at[p, j]],
                       rows.at[p, pl.ds(j * chunk, chunk)], gsem)
                   for j in range(blk)]
            for dd in gds:
                dd.wait()
            for j in range(blk):
                pltpu.async_copy(rows.at[p, pl.ds(j * chunk, chunk)],
                                 acc.at[idx_d.at[p, j]], ssem, add=True)

        emit_block(0, 0, False)
        emit_block(1, 1, False)

        def loop_body(b, carry):
            emit_block(b, lax.rem(b, 2), True)
            return carry

        lax.fori_loop(2, nblk, loop_body, 0)
        for p in range(2):
            pltpu.make_async_copy(gi.at[0, pl.ds(0, blk * chunk)],
                                  rows.at[p], ssem).wait()
        plsc.subcore_barrier()
        for q in range(RPT // STG):
            pltpu.sync_copy(acc.at[pl.ds(r0 + q * STG, STG)], stage)
            pltpu.sync_copy(stage, so.at[c, pl.ds(r0 + q * STG, STG)])
    return body


@functools.lru_cache(maxsize=None)
def _make_edge_pass(D, edge_split, blk=EBLK, chunk=ECHUNK):
    return pl.kernel(
        _edge_pass_body(D, edge_split, blk, chunk),
        out_type=jax.ShapeDtypeStruct((NC, NP, D), jnp.float32),
        mesh=_mesh(),
        scratch_types=[
            pltpu.VMEM((2, blk, chunk), jnp.int32),
            pltpu.VMEM((2, blk, chunk), jnp.int32),
            pltpu.VMEM((2, blk * chunk, D), jnp.float32),
            pltpu.VMEM((STG, D), jnp.float32),
            pltpu.VMEM_SHARED((NP, D), jnp.float32),
            pltpu.SemaphoreType.DMA,
            pltpu.SemaphoreType.DMA,
        ],
        compiler_params=pltpu.CompilerParams(use_tc_tiling_on_sc=False),
    )


def _prep_body(deg_ref, x_ref, dinv_ref, g1i_ref):
    d = deg_ref[0] + deg_ref[1] + 1.0
    dinv = lax.rsqrt(jnp.maximum(d, 1.0))
    dinv_ref[...] = dinv
    g1i_ref[0] = jnp.concatenate(
        [x_ref[...] * dinv, jnp.zeros((RB, DF - IN_CH), jnp.float32)],
        axis=1)
    g1i_ref[1] = jnp.zeros((RB, DF), jnp.float32)


_prep = pl.pallas_call(
    _prep_body,
    grid=(GRID,),
    in_specs=[pl.BlockSpec((NC, RB, 1), lambda i: (0, i, 0)),
              pl.BlockSpec((RB, IN_CH), lambda i: (i, 0))],
    out_specs=[pl.BlockSpec((RB, 1), lambda i: (i, 0)),
               pl.BlockSpec((NC, RB, DF), lambda i: (0, i, 0))],
    out_shape=[jax.ShapeDtypeStruct((NP, 1), jnp.float32),
               jax.ShapeDtypeStruct((NC, NP, DF), jnp.float32)],
)


def _layer_body(mode):
    def body(*refs):
        if mode == "sum":
            s_ref, dinv_ref, w_ref, b_ref, ga_ref, gb_ref = refs
            sv = (s_ref[0] + s_ref[1])[:, :IN_CH]
        else:
            sa_ref, sb_ref, dinv_ref, w_ref, b_ref, ga_ref, gb_ref = refs
            sv = jnp.concatenate([sa_ref[0], sa_ref[1], sb_ref[0],
                                  sb_ref[1]], axis=1)
        dinv = dinv_ref[...]
        h = jnp.dot(sv * dinv, w_ref[...], precision=_HIGH,
                    preferred_element_type=jnp.float32) + b_ref[...]
        h = jnp.maximum(h, 0.0)
        gv = h * dinv
        ga_ref[0] = gv[:, 0 * DF:1 * DF]
        ga_ref[1] = gv[:, 1 * DF:2 * DF]
        gb_ref[0] = gv[:, 2 * DF:3 * DF]
        gb_ref[1] = gv[:, 3 * DF:4 * DF]
    return body


def _make_layer(mode, din):
    s_specs = ([pl.BlockSpec((NC, RB, DF), lambda i: (0, i, 0))]
               if mode == "sum" else
               [pl.BlockSpec((NC, RB, DF), lambda i: (0, i, 0)),
                pl.BlockSpec((NC, RB, DF), lambda i: (0, i, 0))])
    return pl.pallas_call(
        _layer_body(mode),
        grid=(GRID,),
        in_specs=s_specs + [
            pl.BlockSpec((RB, 1), lambda i: (i, 0)),
            pl.BlockSpec((din, H), lambda i: (0, 0)),
            pl.BlockSpec((1, H), lambda i: (0, 0))],
        out_specs=[pl.BlockSpec((NC, RB, DF), lambda i: (0, i, 0)),
                   pl.BlockSpec((NC, RB, DF), lambda i: (0, i, 0))],
        out_shape=[jax.ShapeDtypeStruct((NC, NP, DF), jnp.float32),
                   jax.ShapeDtypeStruct((NC, NP, DF), jnp.float32)],
    )


_layer1 = _make_layer("sum", IN_CH)
_layer2 = _make_layer("concat", H)


def _final_body(sa_ref, sb_ref, dinv_ref, w_ref, b_ref, batch_ref, ge_ref,
                wl_ref, bl_ref, out_ref, pooled, counts):
    i = pl.program_id(0)

    @pl.when(i == 0)
    def _():
        pooled[...] = jnp.zeros((G, H), jnp.float32)
        counts[...] = jnp.zeros((G, 1), jnp.float32)

    sv = jnp.concatenate([sa_ref[0], sa_ref[1], sb_ref[0], sb_ref[1]],
                         axis=1)
    h3 = jnp.dot(sv * dinv_ref[...], w_ref[...], precision=_HIGH,
                 preferred_element_type=jnp.float32) + b_ref[...]
    oh = (batch_ref[...] == lax.broadcasted_iota(jnp.int32, (RB, G), 1)
          ).astype(jnp.float32)
    pooled[...] += lax.dot_general(oh, h3, (((0,), (0,)), ((), ())),
                                   precision=_HIGH,
                                   preferred_element_type=jnp.float32)
    counts[...] += lax.dot_general(oh, jnp.ones((RB, 1), jnp.float32),
                                   (((0,), (0,)), ((), ())),
                                   precision=_HIGH,
                                   preferred_element_type=jnp.float32)

    @pl.when(i == GRID - 1)
    def _():
        pm = pooled[...] / jnp.maximum(counts[...], 1.0)
        z = jnp.dot(jnp.concatenate([pm, ge_ref[...]], axis=1), wl_ref[...],
                    precision=_HIGH,
                    preferred_element_type=jnp.float32) + bl_ref[...]
        m = jnp.max(z, axis=1, keepdims=True)
        lse = jnp.log(jnp.sum(jnp.exp(z - m), axis=1, keepdims=True)) + m
        out_ref[...] = z - lse


_final = pl.pallas_call(
    _final_body,
    grid=(GRID,),
    in_specs=[pl.BlockSpec((NC, RB, DF), lambda i: (0, i, 0)),
              pl.BlockSpec((NC, RB, DF), lambda i: (0, i, 0)),
              pl.BlockSpec((RB, 1), lambda i: (i, 0)),
              pl.BlockSpec((H, H), lambda i: (0, 0)),
              pl.BlockSpec((1, H), lambda i: (0, 0)),
              pl.BlockSpec((RB, 1), lambda i: (i, 0)),
              pl.BlockSpec((G, GE), lambda i: (0, 0)),
              pl.BlockSpec((H + GE, C), lambda i: (0, 0)),
              pl.BlockSpec((1, C), lambda i: (0, 0))],
    out_specs=pl.BlockSpec((G, C), lambda i: (0, 0)),
    out_shape=jax.ShapeDtypeStruct((G, C), jnp.float32),
    scratch_shapes=[pltpu.VMEM((G, H), jnp.float32),
                    pltpu.VMEM((G, 1), jnp.float32)],
)


def kernel(x, edge_index, batch, graph_embedding, W1, b1, W2, b2, W3, b3,
           Wl, bl):
    src = edge_index[0].astype(jnp.int32)
    dst = edge_index[1].astype(jnp.int32)
    pad_e = E_PAD - E
    src_p = jnp.concatenate([src, jnp.zeros((pad_e,), jnp.int32)])
    dst_p = jnp.concatenate([dst, jnp.full((pad_e,), N, jnp.int32)])
    src_c = src_p.reshape(E_PAD // ECHUNK, ECHUNK)
    dst_c = dst_p.reshape(E_PAD // ECHUNK, ECHUNK)
    dst_cd = dst_p.reshape(TOT_CHUNKS, CHUNK)
    x_pad = jnp.pad(x, ((0, NP - N), (0, 0)))
    batch_pad = jnp.pad(batch.astype(jnp.int32), (0, NP - N),
                        constant_values=G).reshape(NP, 1)
    deg2 = _deg_pass()(dst_cd)
    dinv, g1i = _prep(deg2.reshape(NC, NP, 1), x_pad)
    pass1 = _make_edge_pass(DF, True)
    passf = _make_edge_pass(DF, False)
    s1 = pass1(src_c, dst_c, g1i)
    g2a, g2b = _layer1(s1, dinv, W1, b1.reshape(1, H))
    s2a = passf(src_c, dst_c, g2a)
    s2b = passf(src_c, dst_c, g2b)
    g3a, g3b = _layer2(s2a, s2b, dinv, W2, b2.reshape(1, H))
    s3a = passf(src_c, dst_c, g3a)
    s3b = passf(src_c, dst_c, g3b)
    return _final(s3a, s3b, dinv, W3, b3.reshape(1, H), batch_pad,
                  graph_embedding, Wl, bl.reshape(1, C))


# speedup vs baseline: 1.3252x; 1.1139x over previous
import functools

import jax
import jax.numpy as jnp
from jax import lax
from jax.experimental import pallas as pl
from jax.experimental.pallas import tpu as pltpu
from jax.experimental.pallas import tpu_sc as plsc

N = 50000
G = 64
H = 64
GE = 64
C = 5
IN_CH = 2

NC = 2
NS = 16
NP = 50176
RPT = NP // NS
STG = 784

CHUNK = 1024
ECHUNK = 128
EBLK = 10
E = 800000
TOT_CHUNKS = 800
E_PAD = TOT_CHUNKS * CHUNK
CPT = TOT_CHUNKS // NS
CPT_HALF = TOT_CHUNKS // (NS * NC)

DF = H // 4

RB = RPT
GRID = NP // RB

_HIGH = lax.Precision.HIGHEST


def _mesh():
    return plsc.VectorSubcoreMesh(core_axis_name="c", subcore_axis_name="s",
                                  num_cores=NC, num_subcores=NS)


def _deg_body(dst_c, deg2, idx_d, ones_v, stage, acc, ssem):
    c = lax.axis_index("c")
    s = lax.axis_index("s")
    r0 = s * RPT
    for j in range(STG // 16):
        stage[pl.ds(j * 16, 16)] = jnp.zeros((16,), jnp.float32)
    for j in range(CHUNK // 16):
        ones_v[pl.ds(j * 16, 16)] = jnp.ones((16,), jnp.float32)
    for q in range(RPT // STG):
        pltpu.sync_copy(stage, acc.at[pl.ds(r0 + q * STG, STG)])
    plsc.subcore_barrier()
    base = c * (TOT_CHUNKS // NC) + s * CPT_HALF

    def emit_block(b, p, drain):
        if drain:
            pltpu.make_async_copy(deg2.at[pl.ds(0, CHUNK)], ones_v,
                                  ssem).wait()
        pltpu.sync_copy(dst_c.at[base + b], idx_d.at[p])
        pltpu.async_copy(ones_v, acc.at[idx_d.at[p]], ssem, add=True)

    emit_block(0, 0, False)
    emit_block(1, 1, False)

    def blk(b, carry):
        emit_block(b, lax.rem(b, 2), True)
        return carry

    lax.fori_loop(2, CPT_HALF, blk, 0)
    for p in range(2):
        pltpu.make_async_copy(deg2.at[pl.ds(0, CHUNK)], ones_v,
                              ssem).wait()
    plsc.subcore_barrier()
    for q in range(RPT // STG):
        pltpu.sync_copy(acc.at[pl.ds(r0 + q * STG, STG)], stage)
        pltpu.sync_copy(stage, deg2.at[pl.ds(c * NP + r0 + q * STG, STG)])


@functools.lru_cache(maxsize=None)
def _deg_pass():
    return pl.kernel(
        _deg_body,
        out_type=jax.ShapeDtypeStruct((NC * NP,), jnp.float32),
        mesh=_mesh(),
        scratch_types=[
            pltpu.VMEM((2, CHUNK), jnp.int32),
            pltpu.VMEM((CHUNK,), jnp.float32),
            pltpu.VMEM((STG,), jnp.float32),
            pltpu.VMEM_SHARED((NP,), jnp.float32),
            pltpu.SemaphoreType.DMA,
        ],
        compiler_params=pltpu.CompilerParams(use_tc_tiling_on_sc=False),
    )


def _edge_pass_body(D, edge_split, blk, chunk):
    def body(src_c, dst_c, gi, so, idx_s, idx_d, rows, stage, acc, gsem,
             ssem, isem):
        c = lax.axis_index("c")
        s = lax.axis_index("s")
        r0 = s * RPT
        for q in range(RPT // STG):
            pltpu.sync_copy(gi.at[c, pl.ds(r0 + q * STG, STG)], stage)
            pltpu.sync_copy(stage, acc.at[pl.ds(r0 + q * STG, STG)])
        plsc.subcore_barrier()
        tot = E_PAD // chunk
        if edge_split:
            nblk = tot // (NS * NC * blk)
            base = c * (tot // NC) + s * nblk * blk
            table = gi.at[0]
        else:
            nblk = tot // (NS * blk)
            base = s * nblk * blk
            table = gi.at[c]

        def fire_idx(b, q):
            row = lax.min(base + b * blk, base + (nblk - 1) * blk)
            pltpu.async_copy(src_c.at[pl.ds(row, blk)], idx_s.at[q], isem)
            pltpu.async_copy(dst_c.at[pl.ds(row, blk)], idx_d.at[q], isem)

        def wait_idx():
            pltpu.make_async_copy(src_c.at[pl.ds(base, blk)],
                                  idx_s.at[0], isem).wait()
            pltpu.make_async_copy(dst_c.at[pl.ds(base, blk)],
                                  idx_d.at[0], isem).wait()

        def drain_rows(p):
            pltpu.make_async_copy(gi.at[0, pl.ds(0, blk * chunk)],
                                  rows.at[p], ssem).wait()

        def emit_block(b, p, q, drain):
            if drain:
                drain_rows(p)
            fire_idx(b + 1, (b + 1) % 4 if isinstance(b, int)
                     else lax.rem(b + 1, 4))
            wait_idx()
            gds = [pltpu.async_copy(
                       table.at[idx_s.at[q, j]],
                       rows.at[p, pl.ds(j * chunk, chunk)], gsem)
                   for j in range(blk)]
            for dd in gds:
                dd.wait()
            for j in range(blk):
                pltpu.async_copy(rows.at[p, pl.ds(j * chunk, chunk)],
                                 acc.at[idx_d.at[q, j]], ssem, add=True)

        fire_idx(0, 0)
        emit_block(0, 0, 0, False)
        emit_block(1, 1, 1, False)

        def loop_body(b, carry):
            emit_block(b, lax.rem(b, 2), lax.rem(b, 4), True)
            return carry

        lax.fori_loop(2, nblk, loop_body, 0)
        for p in range(2):
            drain_rows(p)
        wait_idx()
        plsc.subcore_barrier()
        for q in range(RPT // STG):
            pltpu.sync_copy(acc.at[pl.ds(r0 + q * STG, STG)], stage)
            pltpu.sync_copy(stage, so.at[c, pl.ds(r0 + q * STG, STG)])
    return body


@functools.lru_cache(maxsize=None)
def _make_edge_pass(D, edge_split, blk=EBLK, chunk=ECHUNK):
    return pl.kernel(
        _edge_pass_body(D, edge_split, blk, chunk),
        out_type=jax.ShapeDtypeStruct((NC, NP, D), jnp.float32),
        mesh=_mesh(),
        scratch_types=[
            pltpu.VMEM((4, blk, chunk), jnp.int32),
            pltpu.VMEM((4, blk, chunk), jnp.int32),
            pltpu.VMEM((2, blk * chunk, D), jnp.float32),
            pltpu.VMEM((STG, D), jnp.float32),
            pltpu.VMEM_SHARED((NP, D), jnp.float32),
            pltpu.SemaphoreType.DMA,
            pltpu.SemaphoreType.DMA,
            pltpu.SemaphoreType.DMA,
        ],
        compiler_params=pltpu.CompilerParams(use_tc_tiling_on_sc=False),
    )


def _prep_body(deg_ref, x_ref, dinv_ref, g1i_ref):
    d = deg_ref[0] + deg_ref[1] + 1.0
    dinv = lax.rsqrt(jnp.maximum(d, 1.0))
    dinv_ref[...] = dinv
    g1i_ref[0] = jnp.concatenate(
        [x_ref[...] * dinv, jnp.zeros((RB, DF - IN_CH), jnp.float32)],
        axis=1)
    g1i_ref[1] = jnp.zeros((RB, DF), jnp.float32)


_prep = pl.pallas_call(
    _prep_body,
    grid=(GRID,),
    in_specs=[pl.BlockSpec((NC, RB, 1), lambda i: (0, i, 0)),
              pl.BlockSpec((RB, IN_CH), lambda i: (i, 0))],
    out_specs=[pl.BlockSpec((RB, 1), lambda i: (i, 0)),
               pl.BlockSpec((NC, RB, DF), lambda i: (0, i, 0))],
    out_shape=[jax.ShapeDtypeStruct((NP, 1), jnp.float32),
               jax.ShapeDtypeStruct((NC, NP, DF), jnp.float32)],
)


def _layer_body(mode):
    def body(*refs):
        if mode == "sum":
            s_ref, dinv_ref, w_ref, b_ref, ga_ref, gb_ref = refs
            sv = (s_ref[0] + s_ref[1])[:, :IN_CH]
        else:
            sa_ref, sb_ref, dinv_ref, w_ref, b_ref, ga_ref, gb_ref = refs
            sv = jnp.concatenate([sa_ref[0], sa_ref[1], sb_ref[0],
                                  sb_ref[1]], axis=1)
        dinv = dinv_ref[...]
        h = jnp.dot(sv * dinv, w_ref[...], precision=_HIGH,
                    preferred_element_type=jnp.float32) + b_ref[...]
        h = jnp.maximum(h, 0.0)
        gv = h * dinv
        ga_ref[0] = gv[:, 0 * DF:1 * DF]
        ga_ref[1] = gv[:, 1 * DF:2 * DF]
        gb_ref[0] = gv[:, 2 * DF:3 * DF]
        gb_ref[1] = gv[:, 3 * DF:4 * DF]
    return body


def _make_layer(mode, din):
    s_specs = ([pl.BlockSpec((NC, RB, DF), lambda i: (0, i, 0))]
               if mode == "sum" else
               [pl.BlockSpec((NC, RB, DF), lambda i: (0, i, 0)),
                pl.BlockSpec((NC, RB, DF), lambda i: (0, i, 0))])
    return pl.pallas_call(
        _layer_body(mode),
        grid=(GRID,),
        in_specs=s_specs + [
            pl.BlockSpec((RB, 1), lambda i: (i, 0)),
            pl.BlockSpec((din, H), lambda i: (0, 0)),
            pl.BlockSpec((1, H), lambda i: (0, 0))],
        out_specs=[pl.BlockSpec((NC, RB, DF), lambda i: (0, i, 0)),
                   pl.BlockSpec((NC, RB, DF), lambda i: (0, i, 0))],
        out_shape=[jax.ShapeDtypeStruct((NC, NP, DF), jnp.float32),
                   jax.ShapeDtypeStruct((NC, NP, DF), jnp.float32)],
    )


_layer1 = _make_layer("sum", IN_CH)
_layer2 = _make_layer("concat", H)


def _final_body(sa_ref, sb_ref, dinv_ref, w_ref, b_ref, batch_ref, ge_ref,
                wl_ref, bl_ref, out_ref, pooled, counts):
    i = pl.program_id(0)

    @pl.when(i == 0)
    def _():
        pooled[...] = jnp.zeros((G, H), jnp.float32)
        counts[...] = jnp.zeros((G, 1), jnp.float32)

    sv = jnp.concatenate([sa_ref[0], sa_ref[1], sb_ref[0], sb_ref[1]],
                         axis=1)
    h3 = jnp.dot(sv * dinv_ref[...], w_ref[...], precision=_HIGH,
                 preferred_element_type=jnp.float32) + b_ref[...]
    oh = (batch_ref[...] == lax.broadcasted_iota(jnp.int32, (RB, G), 1)
          ).astype(jnp.float32)
    pooled[...] += lax.dot_general(oh, h3, (((0,), (0,)), ((), ())),
                                   precision=_HIGH,
                                   preferred_element_type=jnp.float32)
    counts[...] += lax.dot_general(oh, jnp.ones((RB, 1), jnp.float32),
                                   (((0,), (0,)), ((), ())),
                                   precision=_HIGH,
                                   preferred_element_type=jnp.float32)

    @pl.when(i == GRID - 1)
    def _():
        pm = pooled[...] / jnp.maximum(counts[...], 1.0)
        z = jnp.dot(jnp.concatenate([pm, ge_ref[...]], axis=1), wl_ref[...],
                    precision=_HIGH,
                    preferred_element_type=jnp.float32) + bl_ref[...]
        m = jnp.max(z, axis=1, keepdims=True)
        lse = jnp.log(jnp.sum(jnp.exp(z - m), axis=1, keepdims=True)) + m
        out_ref[...] = z - lse


_final = pl.pallas_call(
    _final_body,
    grid=(GRID,),
    in_specs=[pl.BlockSpec((NC, RB, DF), lambda i: (0, i, 0)),
              pl.BlockSpec((NC, RB, DF), lambda i: (0, i, 0)),
              pl.BlockSpec((RB, 1), lambda i: (i, 0)),
              pl.BlockSpec((H, H), lambda i: (0, 0)),
              pl.BlockSpec((1, H), lambda i: (0, 0)),
              pl.BlockSpec((RB, 1), lambda i: (i, 0)),
              pl.BlockSpec((G, GE), lambda i: (0, 0)),
              pl.BlockSpec((H + GE, C), lambda i: (0, 0)),
              pl.BlockSpec((1, C), lambda i: (0, 0))],
    out_specs=pl.BlockSpec((G, C), lambda i: (0, 0)),
    out_shape=jax.ShapeDtypeStruct((G, C), jnp.float32),
    scratch_shapes=[pltpu.VMEM((G, H), jnp.float32),
                    pltpu.VMEM((G, 1), jnp.float32)],
)


def kernel(x, edge_index, batch, graph_embedding, W1, b1, W2, b2, W3, b3,
           Wl, bl):
    src = edge_index[0].astype(jnp.int32)
    dst = edge_index[1].astype(jnp.int32)
    pad_e = E_PAD - E
    src_p = jnp.concatenate([src, jnp.zeros((pad_e,), jnp.int32)])
    dst_p = jnp.concatenate([dst, jnp.full((pad_e,), N, jnp.int32)])
    src_c = src_p.reshape(E_PAD // ECHUNK, ECHUNK)
    dst_c = dst_p.reshape(E_PAD // ECHUNK, ECHUNK)
    dst_cd = dst_p.reshape(TOT_CHUNKS, CHUNK)
    x_pad = jnp.pad(x, ((0, NP - N), (0, 0)))
    batch_pad = jnp.pad(batch.astype(jnp.int32), (0, NP - N),
                        constant_values=G).reshape(NP, 1)
    deg2 = _deg_pass()(dst_cd)
    dinv, g1i = _prep(deg2.reshape(NC, NP, 1), x_pad)
    pass1 = _make_edge_pass(DF, True)
    passf = _make_edge_pass(DF, False)
    s1 = pass1(src_c, dst_c, g1i)
    g2a, g2b = _layer1(s1, dinv, W1, b1.reshape(1, H))
    s2a = passf(src_c, dst_c, g2a)
    s2b = passf(src_c, dst_c, g2b)
    g3a, g3b = _layer2(s2a, s2b, dinv, W2, b2.reshape(1, H))
    s3a = passf(src_c, dst_c, g3a)
    s3b = passf(src_c, dst_c, g3b)
    return _final(s3a, s3b, dinv, W3, b3.reshape(1, H), batch_pad,
                  graph_embedding, Wl, bl.reshape(1, C))


# speedup vs baseline: 1.3329x; 1.0058x over previous
import functools

import jax
import jax.numpy as jnp
from jax import lax
from jax.experimental import pallas as pl
from jax.experimental.pallas import tpu as pltpu
from jax.experimental.pallas import tpu_sc as plsc

N = 50000
G = 64
H = 64
GE = 64
C = 5
IN_CH = 2

NC = 2
NS = 16
NP = 50176
RPT = NP // NS
STG = 784

CHUNK = 1024
ECHUNK = 128
EBLK = 10
E = 800000
TOT_CHUNKS = 800
E_PAD = TOT_CHUNKS * CHUNK
CPT = TOT_CHUNKS // NS
CPT_HALF = TOT_CHUNKS // (NS * NC)

DF = H // 4

RB = RPT
GRID = NP // RB

_HIGH = lax.Precision.HIGHEST


def _mesh():
    return plsc.VectorSubcoreMesh(core_axis_name="c", subcore_axis_name="s",
                                  num_cores=NC, num_subcores=NS)


def _deg_body(dst_c, deg2, idx_d, ones_v, stage, acc, ssem):
    c = lax.axis_index("c")
    s = lax.axis_index("s")
    r0 = s * RPT
    for j in range(STG // 16):
        stage[pl.ds(j * 16, 16)] = jnp.zeros((16,), jnp.float32)
    for j in range(CHUNK // 16):
        ones_v[pl.ds(j * 16, 16)] = jnp.ones((16,), jnp.float32)
    for q in range(RPT // STG):
        pltpu.sync_copy(stage, acc.at[pl.ds(r0 + q * STG, STG)])
    plsc.subcore_barrier()
    base = c * (TOT_CHUNKS // NC) + s * CPT_HALF

    def emit_block(b, p, drain):
        if drain:
            pltpu.make_async_copy(deg2.at[pl.ds(0, CHUNK)], ones_v,
                                  ssem).wait()
        pltpu.sync_copy(dst_c.at[base + b], idx_d.at[p])
        pltpu.async_copy(ones_v, acc.at[idx_d.at[p]], ssem, add=True)

    emit_block(0, 0, False)
    emit_block(1, 1, False)

    def blk(b, carry):
        emit_block(b, lax.rem(b, 2), True)
        return carry

    lax.fori_loop(2, CPT_HALF, blk, 0)
    for p in range(2):
        pltpu.make_async_copy(deg2.at[pl.ds(0, CHUNK)], ones_v,
                              ssem).wait()
    plsc.subcore_barrier()
    for q in range(RPT // STG):
        pltpu.sync_copy(acc.at[pl.ds(r0 + q * STG, STG)], stage)
        pltpu.sync_copy(stage, deg2.at[pl.ds(c * NP + r0 + q * STG, STG)])


@functools.lru_cache(maxsize=None)
def _deg_pass():
    return pl.kernel(
        _deg_body,
        out_type=jax.ShapeDtypeStruct((NC * NP,), jnp.float32),
        mesh=_mesh(),
        scratch_types=[
            pltpu.VMEM((2, CHUNK), jnp.int32),
            pltpu.VMEM((CHUNK,), jnp.float32),
            pltpu.VMEM((STG,), jnp.float32),
            pltpu.VMEM_SHARED((NP,), jnp.float32),
            pltpu.SemaphoreType.DMA,
        ],
        compiler_params=pltpu.CompilerParams(use_tc_tiling_on_sc=False),
    )


def _edge_pass_body(D, edge_split, blk, chunk):
    def body(src_c, dst_c, gi, so, idx_s, idx_d, rows, stage, acc, gsem,
             ssem, isem):
        c = lax.axis_index("c")
        s = lax.axis_index("s")
        r0 = s * RPT
        for q in range(RPT // STG):
            pltpu.sync_copy(gi.at[c, pl.ds(r0 + q * STG, STG)], stage)
            pltpu.sync_copy(stage, acc.at[pl.ds(r0 + q * STG, STG)])
        plsc.subcore_barrier()
        tot = E_PAD // chunk
        if edge_split:
            nblk = tot // (NS * NC * blk)
            base = c * (tot // NC) + s * nblk * blk
            table = gi.at[0]
        else:
            nblk = tot // (NS * blk)
            base = s * nblk * blk
            table = gi.at[c]

        def fire_idx(b, q):
            row = lax.min(base + b * blk, base + (nblk - 1) * blk)
            pltpu.async_copy(src_c.at[pl.ds(row, blk)], idx_s.at[q], isem)
            pltpu.async_copy(dst_c.at[pl.ds(row, blk)], idx_d.at[q], isem)

        def wait_idx():
            pltpu.make_async_copy(src_c.at[pl.ds(base, blk)],
                                  idx_s.at[0], isem).wait()
            pltpu.make_async_copy(dst_c.at[pl.ds(base, blk)],
                                  idx_d.at[0], isem).wait()

        def drain_rows(p):
            pltpu.make_async_copy(gi.at[0, pl.ds(0, blk * chunk)],
                                  rows.at[p], ssem).wait()

        def emit_block(b, p, q, drain):
            if drain:
                drain_rows(p)
            fire_idx(b + 1, (b + 1) % 4 if isinstance(b, int)
                     else lax.rem(b + 1, 4))
            wait_idx()
            gds = [pltpu.async_copy(
                       table.at[idx_s.at[q, j]],
                       rows.at[p, pl.ds(j * chunk, chunk)], gsem)
                   for j in range(blk)]
            for j in range(blk):
                gds[j].wait()
                pltpu.async_copy(rows.at[p, pl.ds(j * chunk, chunk)],
                                 acc.at[idx_d.at[q, j]], ssem, add=True)

        fire_idx(0, 0)
        emit_block(0, 0, 0, False)
        emit_block(1, 1, 1, False)

        def loop_body(b, carry):
            emit_block(b, lax.rem(b, 2), lax.rem(b, 4), True)
            return carry

        lax.fori_loop(2, nblk, loop_body, 0)
        for p in range(2):
            drain_rows(p)
        wait_idx()
        plsc.subcore_barrier()
        for q in range(RPT // STG):
            pltpu.sync_copy(acc.at[pl.ds(r0 + q * STG, STG)], stage)
            pltpu.sync_copy(stage, so.at[c, pl.ds(r0 + q * STG, STG)])
    return body


@functools.lru_cache(maxsize=None)
def _make_edge_pass(D, edge_split, blk=EBLK, chunk=ECHUNK):
    return pl.kernel(
        _edge_pass_body(D, edge_split, blk, chunk),
        out_type=jax.ShapeDtypeStruct((NC, NP, D), jnp.float32),
        mesh=_mesh(),
        scratch_types=[
            pltpu.VMEM((4, blk, chunk), jnp.int32),
            pltpu.VMEM((4, blk, chunk), jnp.int32),
            pltpu.VMEM((2, blk * chunk, D), jnp.float32),
            pltpu.VMEM((STG, D), jnp.float32),
            pltpu.VMEM_SHARED((NP, D), jnp.float32),
            pltpu.SemaphoreType.DMA,
            pltpu.SemaphoreType.DMA,
            pltpu.SemaphoreType.DMA,
        ],
        compiler_params=pltpu.CompilerParams(use_tc_tiling_on_sc=False),
    )


def _prep_body(deg_ref, x_ref, dinv_ref, g1i_ref):
    d = deg_ref[0] + deg_ref[1] + 1.0
    dinv = lax.rsqrt(jnp.maximum(d, 1.0))
    dinv_ref[...] = dinv
    g1i_ref[0] = jnp.concatenate(
        [x_ref[...] * dinv, jnp.zeros((RB, DF - IN_CH), jnp.float32)],
        axis=1)
    g1i_ref[1] = jnp.zeros((RB, DF), jnp.float32)


_prep = pl.pallas_call(
    _prep_body,
    grid=(GRID,),
    in_specs=[pl.BlockSpec((NC, RB, 1), lambda i: (0, i, 0)),
              pl.BlockSpec((RB, IN_CH), lambda i: (i, 0))],
    out_specs=[pl.BlockSpec((RB, 1), lambda i: (i, 0)),
               pl.BlockSpec((NC, RB, DF), lambda i: (0, i, 0))],
    out_shape=[jax.ShapeDtypeStruct((NP, 1), jnp.float32),
               jax.ShapeDtypeStruct((NC, NP, DF), jnp.float32)],
)


def _layer_body(mode):
    def body(*refs):
        if mode == "sum":
            s_ref, dinv_ref, w_ref, b_ref, ga_ref, gb_ref = refs
            sv = (s_ref[0] + s_ref[1])[:, :IN_CH]
        else:
            sa_ref, sb_ref, dinv_ref, w_ref, b_ref, ga_ref, gb_ref = refs
            sv = jnp.concatenate([sa_ref[0], sa_ref[1], sb_ref[0],
                                  sb_ref[1]], axis=1)
        dinv = dinv_ref[...]
        h = jnp.dot(sv * dinv, w_ref[...], precision=_HIGH,
                    preferred_element_type=jnp.float32) + b_ref[...]
        h = jnp.maximum(h, 0.0)
        gv = h * dinv
        ga_ref[0] = gv[:, 0 * DF:1 * DF]
        ga_ref[1] = gv[:, 1 * DF:2 * DF]
        gb_ref[0] = gv[:, 2 * DF:3 * DF]
        gb_ref[1] = gv[:, 3 * DF:4 * DF]
    return body


def _make_layer(mode, din):
    s_specs = ([pl.BlockSpec((NC, RB, DF), lambda i: (0, i, 0))]
               if mode == "sum" else
               [pl.BlockSpec((NC, RB, DF), lambda i: (0, i, 0)),
                pl.BlockSpec((NC, RB, DF), lambda i: (0, i, 0))])
    return pl.pallas_call(
        _layer_body(mode),
        grid=(GRID,),
        in_specs=s_specs + [
            pl.BlockSpec((RB, 1), lambda i: (i, 0)),
            pl.BlockSpec((din, H), lambda i: (0, 0)),
            pl.BlockSpec((1, H), lambda i: (0, 0))],
        out_specs=[pl.BlockSpec((NC, RB, DF), lambda i: (0, i, 0)),
                   pl.BlockSpec((NC, RB, DF), lambda i: (0, i, 0))],
        out_shape=[jax.ShapeDtypeStruct((NC, NP, DF), jnp.float32),
                   jax.ShapeDtypeStruct((NC, NP, DF), jnp.float32)],
    )


_layer1 = _make_layer("sum", IN_CH)
_layer2 = _make_layer("concat", H)


def _final_body(sa_ref, sb_ref, dinv_ref, w_ref, b_ref, batch_ref, ge_ref,
                wl_ref, bl_ref, out_ref, pooled, counts):
    i = pl.program_id(0)

    @pl.when(i == 0)
    def _():
        pooled[...] = jnp.zeros((G, H), jnp.float32)
        counts[...] = jnp.zeros((G, 1), jnp.float32)

    sv = jnp.concatenate([sa_ref[0], sa_ref[1], sb_ref[0], sb_ref[1]],
                         axis=1)
    h3 = jnp.dot(sv * dinv_ref[...], w_ref[...], precision=_HIGH,
                 preferred_element_type=jnp.float32) + b_ref[...]
    oh = (batch_ref[...] == lax.broadcasted_iota(jnp.int32, (RB, G), 1)
          ).astype(jnp.float32)
    pooled[...] += lax.dot_general(oh, h3, (((0,), (0,)), ((), ())),
                                   precision=_HIGH,
                                   preferred_element_type=jnp.float32)
    counts[...] += lax.dot_general(oh, jnp.ones((RB, 1), jnp.float32),
                                   (((0,), (0,)), ((), ())),
                                   precision=_HIGH,
                                   preferred_element_type=jnp.float32)

    @pl.when(i == GRID - 1)
    def _():
        pm = pooled[...] / jnp.maximum(counts[...], 1.0)
        z = jnp.dot(jnp.concatenate([pm, ge_ref[...]], axis=1), wl_ref[...],
                    precision=_HIGH,
                    preferred_element_type=jnp.float32) + bl_ref[...]
        m = jnp.max(z, axis=1, keepdims=True)
        lse = jnp.log(jnp.sum(jnp.exp(z - m), axis=1, keepdims=True)) + m
        out_ref[...] = z - lse


_final = pl.pallas_call(
    _final_body,
    grid=(GRID,),
    in_specs=[pl.BlockSpec((NC, RB, DF), lambda i: (0, i, 0)),
              pl.BlockSpec((NC, RB, DF), lambda i: (0, i, 0)),
              pl.BlockSpec((RB, 1), lambda i: (i, 0)),
              pl.BlockSpec((H, H), lambda i: (0, 0)),
              pl.BlockSpec((1, H), lambda i: (0, 0)),
              pl.BlockSpec((RB, 1), lambda i: (i, 0)),
              pl.BlockSpec((G, GE), lambda i: (0, 0)),
              pl.BlockSpec((H + GE, C), lambda i: (0, 0)),
              pl.BlockSpec((1, C), lambda i: (0, 0))],
    out_specs=pl.BlockSpec((G, C), lambda i: (0, 0)),
    out_shape=jax.ShapeDtypeStruct((G, C), jnp.float32),
    scratch_shapes=[pltpu.VMEM((G, H), jnp.float32),
                    pltpu.VMEM((G, 1), jnp.float32)],
)


def kernel(x, edge_index, batch, graph_embedding, W1, b1, W2, b2, W3, b3,
           Wl, bl):
    src = edge_index[0].astype(jnp.int32)
    dst = edge_index[1].astype(jnp.int32)
    pad_e = E_PAD - E
    src_p = jnp.concatenate([src, jnp.zeros((pad_e,), jnp.int32)])
    dst_p = jnp.concatenate([dst, jnp.full((pad_e,), N, jnp.int32)])
    src_c = src_p.reshape(E_PAD // ECHUNK, ECHUNK)
    dst_c = dst_p.reshape(E_PAD // ECHUNK, ECHUNK)
    dst_cd = dst_p.reshape(TOT_CHUNKS, CHUNK)
    x_pad = jnp.pad(x, ((0, NP - N), (0, 0)))
    batch_pad = jnp.pad(batch.astype(jnp.int32), (0, NP - N),
                        constant_values=G).reshape(NP, 1)
    deg2 = _deg_pass()(dst_cd)
    dinv, g1i = _prep(deg2.reshape(NC, NP, 1), x_pad)
    pass1 = _make_edge_pass(DF, True)
    passf = _make_edge_pass(DF, False)
    s1 = pass1(src_c, dst_c, g1i)
    g2a, g2b = _layer1(s1, dinv, W1, b1.reshape(1, H))
    s2a = passf(src_c, dst_c, g2a)
    s2b = passf(src_c, dst_c, g2b)
    g3a, g3b = _layer2(s2a, s2b, dinv, W2, b2.reshape(1, H))
    s3a = passf(src_c, dst_c, g3a)
    s3b = passf(src_c, dst_c, g3b)
    return _final(s3a, s3b, dinv, W3, b3.reshape(1, H), batch_pad,
                  graph_embedding, Wl, bl.reshape(1, C))
